# baseline (speedup 1.0000x reference)
import functools

import jax
import jax.numpy as jnp
from jax import lax
from jax.experimental import pallas as pl
from jax.experimental.pallas import tpu as pltpu
from jax.experimental.pallas import tpu_sc as plsc

DIM = 64
LANES = 16
NVR = DIM // LANES
GSUB = 128
CHUNK = 640
EPS = 1e-5


_GATHER_DNUMS = lax.GatherDimensionNumbers(
    offset_dims=(), collapsed_slice_dims=(0,), start_index_map=(0,))


def _permute(v, idx):
    return lax.gather(v, idx[:, None], _GATHER_DNUMS, slice_sizes=(1,),
                      mode=lax.GatherScatterMode.PROMISE_IN_BOUNDS)


def _allsum(v, perms):
    for p in perms:
        v = v + _permute(v, p)
    return v


def _rsqrt(x):
    i = lax.bitcast_convert_type(x, jnp.int32)
    i = jnp.int32(0x5F3759DF) - lax.shift_right_logical(i, 1)
    y = lax.bitcast_convert_type(i, jnp.float32)
    hx = x * 0.5
    for _ in range(3):
        y = y * (1.5 - hx * y * y)
    return y


@functools.lru_cache(maxsize=None)
def _make_sc_kernel(n_rows, seq):
    info = plsc.get_sparse_core_info()
    nc, ns = info.num_cores, info.num_subcores
    nw = nc * ns
    rows_per_w = n_rows // nw
    n_chunks = rows_per_w // CHUNK
    n_sub = CHUNK // GSUB
    mesh = plsc.VectorSubcoreMesh(core_axis_name="c", subcore_axis_name="s")

    @functools.partial(
        pl.kernel,
        mesh=mesh,
        compiler_params=pltpu.CompilerParams(use_tc_tiling_on_sc=False),
        out_type=jax.ShapeDtypeStruct((n_rows, DIM), jnp.float32),
        scratch_types=[
            pltpu.VMEM((CHUNK,), jnp.int32),
            pltpu.VMEM((CHUNK, DIM), jnp.float32),
            pltpu.VMEM((seq, DIM), jnp.float32),
            pltpu.VMEM((DIM,), jnp.float32),
            pltpu.VMEM((DIM,), jnp.float32),
            pltpu.SemaphoreType.DMA,
        ],
    )
    def k(tok_hbm, table_hbm, pos_hbm, gamma_hbm, beta_hbm, out_hbm,
          idx_v, rows_v, pos_v, gam_v, bet_v, sem):
        wid = lax.axis_index("s") * nc + lax.axis_index("c")
        tbase = wid * rows_per_w
        pltpu.sync_copy(pos_hbm, pos_v)
        pltpu.sync_copy(gamma_hbm, gam_v)
        pltpu.sync_copy(beta_hbm, bet_v)
        gammas = [gam_v[pl.ds(LANES * j, LANES)] for j in range(NVR)]
        betas = [bet_v[pl.ds(LANES * j, LANES)] for j in range(NVR)]
        lane = lax.iota(jnp.int32, LANES)
        perms = [lane ^ d for d in (1, 2, 4, 8)]

        for c in range(n_chunks):
            cbase = tbase + c * CHUNK
            pltpu.sync_copy(tok_hbm.at[pl.ds(cbase, CHUNK)], idx_v)
            cps = [
                pltpu.async_copy(table_hbm.at[idx_v.at[pl.ds(g * GSUB, GSUB)]],
                                 rows_v.at[pl.ds(g * GSUB, GSUB)], sem)
                for g in range(n_sub)
            ]
            for cp in cps:
                cp.wait()

            def row_body(i, carry):
                s_idx = lax.rem(cbase + i, seq)
                xs = [rows_v[i, pl.ds(LANES * j, LANES)]
                      + pos_v[s_idx, pl.ds(LANES * j, LANES)]
                      for j in range(NVR)]
                tot = (xs[0] + xs[1]) + (xs[2] + xs[3])
                mean = _allsum(tot, perms) * (1.0 / DIM)
                cs = [x - mean for x in xs]
                sq = (cs[0] * cs[0] + cs[1] * cs[1]) + (cs[2] * cs[2]
                                                        + cs[3] * cs[3])
                var = _allsum(sq, perms) * (1.0 / DIM)
                rstd = _rsqrt(var + EPS)
                for j in range(NVR):
                    rows_v[i, pl.ds(LANES * j, LANES)] = (
                        cs[j] * rstd * gammas[j] + betas[j])
                return carry

            lax.fori_loop(0, CHUNK, row_body, 0)
            pltpu.sync_copy(rows_v, out_hbm.at[pl.ds(cbase, CHUNK)])

    return k


def kernel(tokens, table, pos, gamma, beta):
    batch, seq = tokens.shape
    n_rows = batch * seq
    tok_flat = tokens.reshape(n_rows).astype(jnp.int32)
    pos_s = pos[0, :seq, :]
    out = _make_sc_kernel(n_rows, seq)(tok_flat, table, pos_s, gamma, beta)
    return out.reshape(batch, seq, DIM)


# speedup vs baseline: 1.3488x; 1.3488x over previous
import functools

import jax
import jax.numpy as jnp
from jax import lax
from jax.experimental import pallas as pl
from jax.experimental.pallas import tpu as pltpu
from jax.experimental.pallas import tpu_sc as plsc

DIM = 64
LANES = 16
NVR = DIM // LANES
GSUB = 128
CHUNK = 640
EPS = 1e-5
UNROLL = 4

_GATHER_DNUMS = lax.GatherDimensionNumbers(
    offset_dims=(), collapsed_slice_dims=(0,), start_index_map=(0,))


def _permute(v, idx):
    return lax.gather(v, idx[:, None], _GATHER_DNUMS, slice_sizes=(1,),
                      mode=lax.GatherScatterMode.PROMISE_IN_BOUNDS)


def _allsum(v, perms):
    for p in perms:
        v = v + _permute(v, p)
    return v


def _rsqrt(x):
    i = lax.bitcast_convert_type(x, jnp.int32)
    i = jnp.int32(0x5F3759DF) - lax.shift_right_logical(i, 1)
    y = lax.bitcast_convert_type(i, jnp.float32)
    hx = x * 0.5
    for _ in range(3):
        y = y * (1.5 - hx * y * y)
    return y


@functools.lru_cache(maxsize=None)
def _make_sc_kernel(n_rows, seq):
    info = plsc.get_sparse_core_info()
    nc, ns = info.num_cores, info.num_subcores
    nw = nc * ns
    rows_per_w = n_rows // nw
    n_chunks = rows_per_w // CHUNK
    n_sub = CHUNK // GSUB
    mesh = plsc.VectorSubcoreMesh(core_axis_name="c", subcore_axis_name="s")

    @functools.partial(
        pl.kernel,
        mesh=mesh,
        compiler_params=pltpu.CompilerParams(use_tc_tiling_on_sc=False),
        out_type=jax.ShapeDtypeStruct((n_rows, DIM), jnp.float32),
        scratch_types=[
            pltpu.VMEM((CHUNK,), jnp.int32),
            pltpu.VMEM((CHUNK,), jnp.int32),
            pltpu.VMEM((CHUNK, DIM), jnp.float32),
            pltpu.VMEM((CHUNK, DIM), jnp.float32),
            pltpu.VMEM((seq * DIM,), jnp.float32),
            pltpu.VMEM((DIM,), jnp.float32),
            pltpu.VMEM((DIM,), jnp.float32),
            pltpu.SemaphoreType.DMA,
            pltpu.SemaphoreType.DMA,
            pltpu.SemaphoreType.DMA,
            pltpu.SemaphoreType.DMA,
        ],
    )
    def k(tok_hbm, table_hbm, pos_hbm, gamma_hbm, beta_hbm, out_hbm,
          idx0, idx1, rows0, rows1, pos_v, gam_v, bet_v,
          gsem0, gsem1, osem0, osem1):
        idx = [idx0, idx1]
        rows = [rows0, rows1]
        gsem = [gsem0, gsem1]
        osem = [osem0, osem1]
        wid = lax.axis_index("s") * nc + lax.axis_index("c")
        tbase = wid * rows_per_w

        pltpu.sync_copy(pos_hbm, pos_v)
        pltpu.sync_copy(gamma_hbm, gam_v)
        pltpu.sync_copy(beta_hbm, bet_v)
        gammas = [gam_v[pl.ds(LANES * j, LANES)] for j in range(NVR)]
        betas = [bet_v[pl.ds(LANES * j, LANES)] for j in range(NVR)]
        lane = lax.iota(jnp.int32, LANES)
        perms = [lane ^ d for d in (1, 2, 4, 8)]

        def fire_gathers(c):
            b = c % 2
            pltpu.sync_copy(tok_hbm.at[pl.ds(tbase + c * CHUNK, CHUNK)],
                            idx[b])
            return [
                pltpu.async_copy(
                    table_hbm.at[idx[b].at[pl.ds(g * GSUB, GSUB)]],
                    rows[b].at[pl.ds(g * GSUB, GSUB)], gsem[b])
                for g in range(n_sub)
            ]

        gcps = {0: fire_gathers(0)}
        ocps = {}
        for c in range(n_chunks):
            b = c % 2
            if c + 1 < n_chunks:
                if c - 1 >= 0:
                    ocps[c - 1].wait()
                gcps[c + 1] = fire_gathers(c + 1)
            for cp in gcps.pop(c):
                cp.wait()

            rows_b = rows[b]
            phase = (c * CHUNK) % seq

            @plsc.parallel_loop(0, CHUNK, unroll=UNROLL)
            def _(i):
                soff = lax.rem(phase + i, seq) * DIM
                xs = [rows_b[i, pl.ds(LANES * j, LANES)]
                      + pos_v[pl.ds(soff + LANES * j, LANES)]
                      for j in range(NVR)]
                tot = (xs[0] + xs[1]) + (xs[2] + xs[3])
                mean = _allsum(tot, perms) * (1.0 / DIM)
                cs = [x - mean for x in xs]
                sq = (cs[0] * cs[0] + cs[1] * cs[1]) + (cs[2] * cs[2]
                                                        + cs[3] * cs[3])
                var = _allsum(sq, perms) * (1.0 / DIM)
                rstd = _rsqrt(var + EPS)
                for j in range(NVR):
                    rows_b[i, pl.ds(LANES * j, LANES)] = (
                        cs[j] * rstd * gammas[j] + betas[j])

            ocps[c] = pltpu.async_copy(
                rows_b, out_hbm.at[pl.ds(tbase + c * CHUNK, CHUNK)], osem[b])
        ocps[n_chunks - 2].wait()
        ocps[n_chunks - 1].wait()

    return k


def kernel(tokens, table, pos, gamma, beta):
    batch, seq = tokens.shape
    n_rows = batch * seq
    tok_flat = tokens.reshape(n_rows).astype(jnp.int32)
    pos_flat = pos[0, :seq, :].reshape(seq * DIM)
    out = _make_sc_kernel(n_rows, seq)(tok_flat, table, pos_flat,
                                       gamma, beta)
    return out.reshape(batch, seq, DIM)


# speedup vs baseline: 1.3632x; 1.0107x over previous
import functools

import jax
import jax.numpy as jnp
from jax import lax
from jax.experimental import pallas as pl
from jax.experimental.pallas import tpu as pltpu
from jax.experimental.pallas import tpu_sc as plsc

DIM = 64
LANES = 16
NVR = DIM // LANES
NB = 16
LOG_NB = 4
EPS = 1e-5
UNROLL = 4

_GATHER_DNUMS = lax.GatherDimensionNumbers(
    offset_dims=(), collapsed_slice_dims=(0,), start_index_map=(0,))


def _permute(v, idx):
    return lax.gather(v, idx[:, None], _GATHER_DNUMS, slice_sizes=(1,),
                      mode=lax.GatherScatterMode.PROMISE_IN_BOUNDS)


def _allsum(v, perms):
    for p in perms:
        v = v + _permute(v, p)
    return v


def _rsqrt(x):
    i = lax.bitcast_convert_type(x, jnp.int32)
    i = jnp.int32(0x5F3759DF) - lax.shift_right_logical(i, 1)
    y = lax.bitcast_convert_type(i, jnp.float32)
    hx = x * 0.5
    for _ in range(3):
        y = y * (1.5 - hx * y * y)
    return y


@functools.lru_cache(maxsize=None)
def _make_sc_kernel(batch, seq):
    info = plsc.get_sparse_core_info()
    nc, ns = info.num_cores, info.num_subcores
    nw = nc * ns
    b_per_w = batch // nw
    n_chunks = b_per_w // NB
    rows = seq * NB
    mesh = plsc.VectorSubcoreMesh(core_axis_name="c", subcore_axis_name="s")

    @functools.partial(
        pl.kernel,
        mesh=mesh,
        compiler_params=pltpu.CompilerParams(use_tc_tiling_on_sc=False),
        out_type=jax.ShapeDtypeStruct((seq, batch, DIM), jnp.float32),
        scratch_types=[
            pltpu.VMEM((seq, NB), jnp.int32),
            pltpu.VMEM((seq, NB), jnp.int32),
            pltpu.VMEM((seq, NB, DIM), jnp.float32),
            pltpu.VMEM((seq, NB, DIM), jnp.float32),
            pltpu.VMEM((seq, DIM), jnp.float32),
            pltpu.VMEM((DIM,), jnp.float32),
            pltpu.VMEM((DIM,), jnp.float32),
            pltpu.SemaphoreType.DMA,
            pltpu.SemaphoreType.DMA,
            pltpu.SemaphoreType.DMA,
            pltpu.SemaphoreType.DMA,
        ],
    )
    def k(tok_hbm, table_hbm, pos_hbm, gamma_hbm, beta_hbm, out_hbm,
          idx0, idx1, rows0, rows1, pos_v, gam_v, bet_v,
          gsem0, gsem1, osem0, osem1):
        idx = [idx0, idx1]
        rbuf = [rows0, rows1]
        gsem = [gsem0, gsem1]
        osem = [osem0, osem1]
        wid = lax.axis_index("s") * nc + lax.axis_index("c")
        bbase = wid * b_per_w

        pltpu.sync_copy(pos_hbm.at[pl.ds(0, seq)], pos_v)
        pltpu.sync_copy(gamma_hbm, gam_v)
        pltpu.sync_copy(beta_hbm, bet_v)
        gammas = [gam_v[pl.ds(LANES * j, LANES)] for j in range(NVR)]
        betas = [bet_v[pl.ds(LANES * j, LANES)] for j in range(NVR)]
        lane = lax.iota(jnp.int32, LANES)
        perms = [lane ^ d for d in (1, 2, 4, 8)]

        def fire_gathers(c):
            b = c % 2
            pltpu.sync_copy(
                tok_hbm.at[:, pl.ds(bbase + c * NB, NB)], idx[b])
            return [
                pltpu.async_copy(table_hbm.at[idx[b].at[s]],
                                 rbuf[b].at[s], gsem[b])
                for s in range(seq)
            ]

        gcps = {0: fire_gathers(0)}
        ocps = {}
        for c in range(n_chunks):
            b = c % 2
            if c + 1 < n_chunks:
                if c - 1 >= 0:
                    ocps[c - 1].wait()
                gcps[c + 1] = fire_gathers(c + 1)
            for cp in gcps.pop(c):
                cp.wait()

            rows_b = rbuf[c % 2]

            @plsc.parallel_loop(0, rows, unroll=UNROLL)
            def _(i):
                s = lax.shift_right_logical(i, LOG_NB)
                r = lax.bitwise_and(i, NB - 1)
                xs = [rows_b[s, r, pl.ds(LANES * j, LANES)]
                      + pos_v[s, pl.ds(LANES * j, LANES)]
                      for j in range(NVR)]
                tot = (xs[0] + xs[1]) + (xs[2] + xs[3])
                mean = _allsum(tot, perms) * (1.0 / DIM)
                cs = [x - mean for x in xs]
                sq = (cs[0] * cs[0] + cs[1] * cs[1]) + (cs[2] * cs[2]
                                                        + cs[3] * cs[3])
                var = _allsum(sq, perms) * (1.0 / DIM)
                rstd = _rsqrt(var + EPS)
                for j in range(NVR):
                    rows_b[s, r, pl.ds(LANES * j, LANES)] = (
                        cs[j] * rstd * gammas[j] + betas[j])

            ocps[c] = pltpu.async_copy(
                rows_b, out_hbm.at[:, pl.ds(bbase + c * NB, NB)],
                osem[c % 2])
        for c in (n_chunks - 2, n_chunks - 1):
            ocps[c].wait()

    return k


def kernel(tokens, table, pos, gamma, beta):
    batch, seq = tokens.shape
    out = _make_sc_kernel(batch, seq)(tokens.T.astype(jnp.int32), table,
                                      pos[0], gamma, beta)
    return out.transpose(1, 0, 2)


# speedup vs baseline: 1.9887x; 1.4589x over previous
import functools

import jax
import jax.numpy as jnp
from jax import lax
from jax.experimental import pallas as pl
from jax.experimental.pallas import tpu as pltpu
from jax.experimental.pallas import tpu_sc as plsc

DIM = 64
LANES = 16
NVR = DIM // LANES
NB = 16
LOG_NB = 4
EPS = 1e-5
UNROLL = 4

_GATHER_DNUMS = lax.GatherDimensionNumbers(
    offset_dims=(), collapsed_slice_dims=(0,), start_index_map=(0,))


def _permute(v, idx):
    return lax.gather(v, idx[:, None], _GATHER_DNUMS, slice_sizes=(1,),
                      mode=lax.GatherScatterMode.PROMISE_IN_BOUNDS)


def _allsum(v, perms):
    for p in perms:
        v = v + _permute(v, p)
    return v


def _rsqrt(x):
    i = lax.bitcast_convert_type(x, jnp.int32)
    i = jnp.int32(0x5F3759DF) - lax.shift_right_logical(i, 1)
    y = lax.bitcast_convert_type(i, jnp.float32)
    hx = x * 0.5
    for _ in range(3):
        y = y * (1.5 - hx * y * y)
    return y


_VB = 4096
_LOG_VB = 12


@functools.lru_cache(maxsize=None)
def _make_retile(vocab):
    grid = vocab // _VB

    def body(in_ref, out_ref):
        z = in_ref[...].T
        out_ref[:, 0:DIM] = z[0:_VB // 2]
        out_ref[:, DIM:2 * DIM] = z[_VB // 2:_VB]

    return pl.pallas_call(
        body,
        grid=(grid,),
        in_specs=[pl.BlockSpec((DIM, _VB), lambda i: (0, i))],
        out_specs=pl.BlockSpec((_VB // 2, 2 * DIM), lambda i: (i, 0)),
        out_shape=jax.ShapeDtypeStruct((vocab // 2, 2 * DIM), jnp.float32),
    )


@functools.lru_cache(maxsize=None)
def _make_tail(vocab):
    start = (vocab // _VB) * _VB
    n = vocab - start
    half = n // 2
    qrow = start // 2

    def body(tail_ref, acc_ref, out_ref, vout, sem):
        del acc_ref
        z = tail_ref[...].T
        vout[:, 0:DIM] = z[0:half]
        vout[:, DIM:2 * DIM] = z[half:n]
        cp = pltpu.make_async_copy(
            vout, out_ref.at[pl.ds(qrow, half)], sem)
        cp.start()
        cp.wait()

    return pl.pallas_call(
        body,
        in_specs=[pl.BlockSpec((DIM, n), lambda: (0, 0)),
                  pl.BlockSpec(memory_space=pl.ANY)],
        out_specs=pl.BlockSpec(memory_space=pl.ANY),
        scratch_shapes=[pltpu.VMEM((half, 2 * DIM), jnp.float32),
                        pltpu.SemaphoreType.DMA],
        out_shape=jax.ShapeDtypeStruct((vocab // 2, 2 * DIM), jnp.float32),
        input_output_aliases={1: 0},
    )


@functools.lru_cache(maxsize=None)
def _make_sc_kernel(batch, seq, vocab):
    tstart = (vocab // _VB) * _VB
    thalf = (vocab - tstart) // 2
    info = plsc.get_sparse_core_info()
    nc, ns = info.num_cores, info.num_subcores
    nw = nc * ns
    b_per_w = batch // nw
    n_chunks = b_per_w // NB
    rows = seq * NB
    mesh = plsc.VectorSubcoreMesh(core_axis_name="c", subcore_axis_name="s")

    @functools.partial(
        pl.kernel,
        mesh=mesh,
        compiler_params=pltpu.CompilerParams(use_tc_tiling_on_sc=False),
        out_type=jax.ShapeDtypeStruct((seq, batch, DIM), jnp.float32),
        scratch_types=[
            pltpu.VMEM((seq, NB), jnp.int32),
            pltpu.VMEM((seq, NB), jnp.int32),
            pltpu.VMEM((seq, NB, DIM), jnp.float32),
            pltpu.VMEM((seq, NB, DIM), jnp.float32),
            pltpu.VMEM((seq, DIM), jnp.float32),
            pltpu.VMEM((DIM,), jnp.float32),
            pltpu.VMEM((DIM,), jnp.float32),
            pltpu.SemaphoreType.DMA,
            pltpu.SemaphoreType.DMA,
            pltpu.SemaphoreType.DMA,
            pltpu.SemaphoreType.DMA,
        ],
    )
    def k(tok_hbm, table_hbm, pos_hbm, gamma_hbm, beta_hbm, out_hbm,
          idx0, idx1, rows0, rows1, pos_v, gam_v, bet_v,
          gsem0, gsem1, osem0, osem1):
        idx = [idx0, idx1]
        rbuf = [rows0, rows1]
        gsem = [gsem0, gsem1]
        osem = [osem0, osem1]
        wid = lax.axis_index("s") * nc + lax.axis_index("c")
        bbase = wid * b_per_w

        pltpu.sync_copy(pos_hbm.at[pl.ds(0, seq)], pos_v)
        pltpu.sync_copy(gamma_hbm, gam_v)
        pltpu.sync_copy(beta_hbm, bet_v)
        gammas = [gam_v[pl.ds(LANES * j, LANES)] for j in range(NVR)]
        betas = [bet_v[pl.ds(LANES * j, LANES)] for j in range(NVR)]
        lane = lax.iota(jnp.int32, LANES)
        perms = [lane ^ d for d in (1, 2, 4, 8)]

        def fire_gathers(c):
            b = c % 2
            pltpu.sync_copy(
                tok_hbm.at[:, pl.ds(bbase + c * NB, NB)], idx[b])
            for s in range(seq):
                v = idx[b][s, :]
                q = ((v & ~(_VB - 1))
                     + ((v & (_VB // 2 - 1)) << 1)
                     + ((v >> (_LOG_VB - 1)) & 1))
                if tstart < vocab:
                    w = v - tstart
                    hi = w >= thalf
                    qt = (tstart + ((w - jnp.where(hi, thalf, 0)) << 1)
                          + jnp.where(hi, 1, 0))
                    q = jnp.where(v >= tstart, qt, q)
                idx[b][s, :] = q
            return [
                pltpu.async_copy(table_hbm.at[idx[b].at[s]],
                                 rbuf[b].at[s], gsem[b])
                for s in range(seq)
            ]

        gcps = {0: fire_gathers(0)}
        ocps = {}
        for c in range(n_chunks):
            b = c % 2
            if c + 1 < n_chunks:
                if c - 1 >= 0:
                    ocps[c - 1].wait()
                gcps[c + 1] = fire_gathers(c + 1)
            for cp in gcps.pop(c):
                cp.wait()

            rows_b = rbuf[c % 2]

            @plsc.parallel_loop(0, rows, unroll=UNROLL)
            def _(i):
                s = lax.shift_right_logical(i, LOG_NB)
                r = lax.bitwise_and(i, NB - 1)
                xs = [rows_b[s, r, pl.ds(LANES * j, LANES)]
                      + pos_v[s, pl.ds(LANES * j, LANES)]
                      for j in range(NVR)]
                tot = (xs[0] + xs[1]) + (xs[2] + xs[3])
                mean = _allsum(tot, perms) * (1.0 / DIM)
                cs = [x - mean for x in xs]
                sq = (cs[0] * cs[0] + cs[1] * cs[1]) + (cs[2] * cs[2]
                                                        + cs[3] * cs[3])
                var = _allsum(sq, perms) * (1.0 / DIM)
                rstd = _rsqrt(var + EPS)
                for j in range(NVR):
                    rows_b[s, r, pl.ds(LANES * j, LANES)] = (
                        cs[j] * rstd * gammas[j] + betas[j])

            ocps[c] = pltpu.async_copy(
                rows_b, out_hbm.at[:, pl.ds(bbase + c * NB, NB)],
                osem[c % 2])
        for c in (n_chunks - 2, n_chunks - 1):
            ocps[c].wait()

    return k


def kernel(tokens, table, pos, gamma, beta):
    batch, seq = tokens.shape
    vocab = table.shape[0]
    tab_t = table.T
    tab2 = _make_retile(vocab)(tab_t)
    if vocab % _VB:
        tab2 = _make_tail(vocab)(tab_t[:, (vocab // _VB) * _VB:], tab2)
    tab_lin = tab2.reshape(vocab, DIM)
    out = _make_sc_kernel(batch, seq, vocab)(tokens.T.astype(jnp.int32),
                                             tab_lin, pos[0], gamma, beta)
    return out.transpose(1, 0, 2)


# speedup vs baseline: 2.2542x; 1.1335x over previous
import functools

import jax
import jax.numpy as jnp
from jax import lax
from jax.experimental import pallas as pl
from jax.experimental.pallas import tpu as pltpu
from jax.experimental.pallas import tpu_sc as plsc

DIM = 64
LANES = 16
NVR = DIM // LANES
NB = 16
LOG_NB = 4
EPS = 1e-5
UNROLL = 4

_GATHER_DNUMS = lax.GatherDimensionNumbers(
    offset_dims=(), collapsed_slice_dims=(0,), start_index_map=(0,))


def _permute(v, idx):
    return lax.gather(v, idx[:, None], _GATHER_DNUMS, slice_sizes=(1,),
                      mode=lax.GatherScatterMode.PROMISE_IN_BOUNDS)


def _allsum(v, perms):
    for p in perms:
        v = v + _permute(v, p)
    return v


def _rsqrt(x):
    i = lax.bitcast_convert_type(x, jnp.int32)
    i = jnp.int32(0x5F3759DF) - lax.shift_right_logical(i, 1)
    y = lax.bitcast_convert_type(i, jnp.float32)
    hx = x * 0.5
    for _ in range(3):
        y = y * (1.5 - hx * y * y)
    return y


_VB = 8192
_LOG_VB = 13


@functools.lru_cache(maxsize=None)
def _make_retile(vocab):
    grid = vocab // _VB

    def body(in_ref, out_ref):
        z = in_ref[...].T
        out_ref[:, 0:DIM] = z[0:_VB // 2]
        out_ref[:, DIM:2 * DIM] = z[_VB // 2:_VB]

    return pl.pallas_call(
        body,
        grid=(grid,),
        in_specs=[pl.BlockSpec((DIM, _VB), lambda i: (0, i))],
        out_specs=pl.BlockSpec((_VB // 2, 2 * DIM), lambda i: (i, 0)),
        out_shape=jax.ShapeDtypeStruct((vocab // 2, 2 * DIM), jnp.float32),
    )


@functools.lru_cache(maxsize=None)
def _make_tail(vocab):
    start = (vocab // _VB) * _VB
    n = vocab - start
    half = n // 2
    qrow = start // 2

    def body(tail_ref, acc_ref, out_ref, vout, sem):
        del acc_ref
        z = tail_ref[...].T
        vout[:, 0:DIM] = z[0:half]
        vout[:, DIM:2 * DIM] = z[half:n]
        cp = pltpu.make_async_copy(
            vout, out_ref.at[pl.ds(qrow, half)], sem)
        cp.start()
        cp.wait()

    return pl.pallas_call(
        body,
        in_specs=[pl.BlockSpec((DIM, n), lambda: (0, 0)),
                  pl.BlockSpec(memory_space=pl.ANY)],
        out_specs=pl.BlockSpec(memory_space=pl.ANY),
        scratch_shapes=[pltpu.VMEM((half, 2 * DIM), jnp.float32),
                        pltpu.SemaphoreType.DMA],
        out_shape=jax.ShapeDtypeStruct((vocab // 2, 2 * DIM), jnp.float32),
        input_output_aliases={1: 0},
    )


@functools.lru_cache(maxsize=None)
def _make_sc_kernel(batch, seq, vocab):
    tstart = (vocab // _VB) * _VB
    thalf = (vocab - tstart) // 2
    info = plsc.get_sparse_core_info()
    nc, ns = info.num_cores, info.num_subcores
    nw = nc * ns
    b_per_w = batch // nw
    n_chunks = b_per_w // NB
    rows = seq * NB
    mesh = plsc.VectorSubcoreMesh(core_axis_name="c", subcore_axis_name="s")

    @functools.partial(
        pl.kernel,
        mesh=mesh,
        compiler_params=pltpu.CompilerParams(use_tc_tiling_on_sc=False),
        out_type=jax.ShapeDtypeStruct((seq, batch, DIM), jnp.float32),
        scratch_types=[
            pltpu.VMEM((seq, NB), jnp.int32),
            pltpu.VMEM((seq, NB), jnp.int32),
            pltpu.VMEM((seq, NB, DIM), jnp.float32),
            pltpu.VMEM((seq, NB, DIM), jnp.float32),
            pltpu.VMEM((seq, DIM), jnp.float32),
            pltpu.VMEM((DIM,), jnp.float32),
            pltpu.VMEM((DIM,), jnp.float32),
            pltpu.SemaphoreType.DMA,
            pltpu.SemaphoreType.DMA,
            pltpu.SemaphoreType.DMA,
            pltpu.SemaphoreType.DMA,
        ],
    )
    def k(tok_hbm, table_hbm, pos_hbm, gamma_hbm, beta_hbm, out_hbm,
          idx0, idx1, rows0, rows1, pos_v, gam_v, bet_v,
          gsem0, gsem1, osem0, osem1):
        idx = [idx0, idx1]
        rbuf = [rows0, rows1]
        gsem = [gsem0, gsem1]
        osem = [osem0, osem1]
        wid = lax.axis_index("s") * nc + lax.axis_index("c")
        bbase = wid * b_per_w

        pltpu.sync_copy(pos_hbm.at[pl.ds(0, seq)], pos_v)
        pltpu.sync_copy(gamma_hbm, gam_v)
        pltpu.sync_copy(beta_hbm, bet_v)
        gammas = [gam_v[pl.ds(LANES * j, LANES)] for j in range(NVR)]
        betas = [bet_v[pl.ds(LANES * j, LANES)] for j in range(NVR)]
        lane = lax.iota(jnp.int32, LANES)
        perms = [lane ^ d for d in (1, 2, 4, 8)]

        def fire_gathers(c):
            b = c % 2
            pltpu.sync_copy(
                tok_hbm.at[:, pl.ds(bbase + c * NB, NB)], idx[b])
            for s in range(seq):
                v = idx[b][s, :]
                q = ((v & ~(_VB - 1))
                     + ((v & (_VB // 2 - 1)) << 1)
                     + ((v >> (_LOG_VB - 1)) & 1))
                if tstart < vocab:
                    w = v - tstart
                    hi = w >= thalf
                    qt = (tstart + ((w - jnp.where(hi, thalf, 0)) << 1)
                          + jnp.where(hi, 1, 0))
                    q = jnp.where(v >= tstart, qt, q)
                idx[b][s, :] = q
            return [
                pltpu.async_copy(table_hbm.at[idx[b].at[s]],
                                 rbuf[b].at[s], gsem[b])
                for s in range(seq)
            ]

        gcps = {0: fire_gathers(0)}
        ocps = {}
        for c in range(n_chunks):
            b = c % 2
            if c + 1 < n_chunks:
                if c - 1 >= 0:
                    ocps[c - 1].wait()
                gcps[c + 1] = fire_gathers(c + 1)
            for cp in gcps.pop(c):
                cp.wait()

            rows_b = rbuf[c % 2]

            @plsc.parallel_loop(0, rows, unroll=UNROLL)
            def _(i):
                s = lax.shift_right_logical(i, LOG_NB)
                r = lax.bitwise_and(i, NB - 1)
                xs = [rows_b[s, r, pl.ds(LANES * j, LANES)]
                      + pos_v[s, pl.ds(LANES * j, LANES)]
                      for j in range(NVR)]
                tot = (xs[0] + xs[1]) + (xs[2] + xs[3])
                mean = _allsum(tot, perms) * (1.0 / DIM)
                cs = [x - mean for x in xs]
                sq = (cs[0] * cs[0] + cs[1] * cs[1]) + (cs[2] * cs[2]
                                                        + cs[3] * cs[3])
                var = _allsum(sq, perms) * (1.0 / DIM)
                rstd = _rsqrt(var + EPS)
                for j in range(NVR):
                    rows_b[s, r, pl.ds(LANES * j, LANES)] = (
                        cs[j] * rstd * gammas[j] + betas[j])

            ocps[c] = pltpu.async_copy(
                rows_b, out_hbm.at[:, pl.ds(bbase + c * NB, NB)],
                osem[c % 2])
        for c in (n_chunks - 2, n_chunks - 1):
            ocps[c].wait()

    return k


def kernel(tokens, table, pos, gamma, beta):
    batch, seq = tokens.shape
    vocab = table.shape[0]
    tab_t = table.T
    tab2 = _make_retile(vocab)(tab_t)
    if vocab % _VB:
        tab2 = _make_tail(vocab)(tab_t[:, (vocab // _VB) * _VB:], tab2)
    tab_lin = tab2.reshape(vocab, DIM)
    out = _make_sc_kernel(batch, seq, vocab)(tokens.T.astype(jnp.int32),
                                             tab_lin, pos[0], gamma, beta)
    return out.transpose(1, 0, 2)


# speedup vs baseline: 2.4134x; 1.0706x over previous
import functools

import jax
import jax.numpy as jnp
from jax import lax
from jax.experimental import pallas as pl
from jax.experimental.pallas import tpu as pltpu
from jax.experimental.pallas import tpu_sc as plsc

DIM = 64
LANES = 16
NVR = DIM // LANES
NB = 16
LOG_NB = 4
EPS = 1e-5
UNROLL = 4

_GATHER_DNUMS = lax.GatherDimensionNumbers(
    offset_dims=(), collapsed_slice_dims=(0,), start_index_map=(0,))


def _permute(v, idx):
    return lax.gather(v, idx[:, None], _GATHER_DNUMS, slice_sizes=(1,),
                      mode=lax.GatherScatterMode.PROMISE_IN_BOUNDS)


def _allsum(v, perms):
    for p in perms:
        v = v + _permute(v, p)
    return v


def _rsqrt(x):
    i = lax.bitcast_convert_type(x, jnp.int32)
    i = jnp.int32(0x5F3759DF) - lax.shift_right_logical(i, 1)
    y = lax.bitcast_convert_type(i, jnp.float32)
    hx = x * 0.5
    for _ in range(3):
        y = y * (1.5 - hx * y * y)
    return y


_VB = 16384
_LOG_VB = 14


@functools.lru_cache(maxsize=None)
def _make_retile(vocab):
    grid = vocab // _VB

    def body(in_ref, out_ref):
        z = in_ref[...].T
        out_ref[:, 0:DIM] = z[0:_VB // 2]
        out_ref[:, DIM:2 * DIM] = z[_VB // 2:_VB]

    return pl.pallas_call(
        body,
        grid=(grid,),
        in_specs=[pl.BlockSpec((DIM, _VB), lambda i: (0, i))],
        out_specs=pl.BlockSpec((_VB // 2, 2 * DIM), lambda i: (i, 0)),
        out_shape=jax.ShapeDtypeStruct((vocab // 2, 2 * DIM), jnp.float32),
    )


@functools.lru_cache(maxsize=None)
def _make_tail(vocab):
    start = (vocab // _VB) * _VB
    n = vocab - start
    half = n // 2
    qrow = start // 2

    def body(tail_ref, acc_ref, out_ref, vout, sem):
        del acc_ref
        z = tail_ref[...].T
        vout[:, 0:DIM] = z[0:half]
        vout[:, DIM:2 * DIM] = z[half:n]
        cp = pltpu.make_async_copy(
            vout, out_ref.at[pl.ds(qrow, half)], sem)
        cp.start()
        cp.wait()

    return pl.pallas_call(
        body,
        in_specs=[pl.BlockSpec((DIM, n), lambda: (0, 0)),
                  pl.BlockSpec(memory_space=pl.ANY)],
        out_specs=pl.BlockSpec(memory_space=pl.ANY),
        scratch_shapes=[pltpu.VMEM((half, 2 * DIM), jnp.float32),
                        pltpu.SemaphoreType.DMA],
        out_shape=jax.ShapeDtypeStruct((vocab // 2, 2 * DIM), jnp.float32),
        input_output_aliases={1: 0},
    )


@functools.lru_cache(maxsize=None)
def _make_sc_kernel(batch, seq, vocab):
    tstart = (vocab // _VB) * _VB
    thalf = (vocab - tstart) // 2
    info = plsc.get_sparse_core_info()
    nc, ns = info.num_cores, info.num_subcores
    nw = nc * ns
    b_per_w = batch // nw
    n_chunks = b_per_w // NB
    rows = seq * NB
    mesh = plsc.VectorSubcoreMesh(core_axis_name="c", subcore_axis_name="s")

    @functools.partial(
        pl.kernel,
        mesh=mesh,
        compiler_params=pltpu.CompilerParams(use_tc_tiling_on_sc=False),
        out_type=jax.ShapeDtypeStruct((seq, batch, DIM), jnp.float32),
        scratch_types=[
            pltpu.VMEM((seq, NB), jnp.int32),
            pltpu.VMEM((seq, NB), jnp.int32),
            pltpu.VMEM((seq, NB, DIM), jnp.float32),
            pltpu.VMEM((seq, NB, DIM), jnp.float32),
            pltpu.VMEM((seq, DIM), jnp.float32),
            pltpu.VMEM((DIM,), jnp.float32),
            pltpu.VMEM((DIM,), jnp.float32),
            pltpu.SemaphoreType.DMA,
            pltpu.SemaphoreType.DMA,
            pltpu.SemaphoreType.DMA,
            pltpu.SemaphoreType.DMA,
        ],
    )
    def k(tok_hbm, table_hbm, pos_hbm, gamma_hbm, beta_hbm, out_hbm,
          idx0, idx1, rows0, rows1, pos_v, gam_v, bet_v,
          gsem0, gsem1, osem0, osem1):
        idx = [idx0, idx1]
        rbuf = [rows0, rows1]
        gsem = [gsem0, gsem1]
        osem = [osem0, osem1]
        wid = lax.axis_index("s") * nc + lax.axis_index("c")
        bbase = wid * b_per_w

        pltpu.sync_copy(pos_hbm.at[pl.ds(0, seq)], pos_v)
        pltpu.sync_copy(gamma_hbm, gam_v)
        pltpu.sync_copy(beta_hbm, bet_v)
        gammas = [gam_v[pl.ds(LANES * j, LANES)] for j in range(NVR)]
        betas = [bet_v[pl.ds(LANES * j, LANES)] for j in range(NVR)]
        lane = lax.iota(jnp.int32, LANES)
        perms = [lane ^ d for d in (1, 2, 4, 8)]

        def fire_gathers(c):
            b = c % 2
            pltpu.sync_copy(
                tok_hbm.at[:, pl.ds(bbase + c * NB, NB)], idx[b])
            for s in range(seq):
                v = idx[b][s, :]
                q = ((v & ~(_VB - 1))
                     + ((v & (_VB // 2 - 1)) << 1)
                     + ((v >> (_LOG_VB - 1)) & 1))
                if tstart < vocab:
                    w = v - tstart
                    hi = w >= thalf
                    qt = (tstart + ((w - jnp.where(hi, thalf, 0)) << 1)
                          + jnp.where(hi, 1, 0))
                    q = jnp.where(v >= tstart, qt, q)
                idx[b][s, :] = q
            return [
                pltpu.async_copy(table_hbm.at[idx[b].at[s]],
                                 rbuf[b].at[s], gsem[b])
                for s in range(seq)
            ]

        gcps = {0: fire_gathers(0)}
        ocps = {}
        for c in range(n_chunks):
            b = c % 2
            if c + 1 < n_chunks:
                if c - 1 >= 0:
                    ocps[c - 1].wait()
                gcps[c + 1] = fire_gathers(c + 1)
            for cp in gcps.pop(c):
                cp.wait()

            rows_b = rbuf[c % 2]

            @plsc.parallel_loop(0, rows, unroll=UNROLL)
            def _(i):
                s = lax.shift_right_logical(i, LOG_NB)
                r = lax.bitwise_and(i, NB - 1)
                xs = [rows_b[s, r, pl.ds(LANES * j, LANES)]
                      + pos_v[s, pl.ds(LANES * j, LANES)]
                      for j in range(NVR)]
                tot = (xs[0] + xs[1]) + (xs[2] + xs[3])
                mean = _allsum(tot, perms) * (1.0 / DIM)
                cs = [x - mean for x in xs]
                sq = (cs[0] * cs[0] + cs[1] * cs[1]) + (cs[2] * cs[2]
                                                        + cs[3] * cs[3])
                var = _allsum(sq, perms) * (1.0 / DIM)
                rstd = _rsqrt(var + EPS)
                for j in range(NVR):
                    rows_b[s, r, pl.ds(LANES * j, LANES)] = (
                        cs[j] * rstd * gammas[j] + betas[j])

            ocps[c] = pltpu.async_copy(
                rows_b, out_hbm.at[:, pl.ds(bbase + c * NB, NB)],
                osem[c % 2])
        for c in (n_chunks - 2, n_chunks - 1):
            ocps[c].wait()

    return k


def kernel(tokens, table, pos, gamma, beta):
    batch, seq = tokens.shape
    vocab = table.shape[0]
    tab_t = table.T
    tab2 = _make_retile(vocab)(tab_t)
    if vocab % _VB:
        tab2 = _make_tail(vocab)(tab_t[:, (vocab // _VB) * _VB:], tab2)
    tab_lin = tab2.reshape(vocab, DIM)
    out = _make_sc_kernel(batch, seq, vocab)(tokens.T.astype(jnp.int32),
                                             tab_lin, pos[0], gamma, beta)
    return out.transpose(1, 0, 2)


# speedup vs baseline: 2.4641x; 1.0210x over previous
import functools

import jax
import jax.numpy as jnp
from jax import lax
from jax.experimental import pallas as pl
from jax.experimental.pallas import tpu as pltpu
from jax.experimental.pallas import tpu_sc as plsc

DIM = 64
LANES = 16
NVR = DIM // LANES
NB = 16
LOG_NB = 4
EPS = 1e-5
UNROLL = 4

_GATHER_DNUMS = lax.GatherDimensionNumbers(
    offset_dims=(), collapsed_slice_dims=(0,), start_index_map=(0,))


def _permute(v, idx):
    return lax.gather(v, idx[:, None], _GATHER_DNUMS, slice_sizes=(1,),
                      mode=lax.GatherScatterMode.PROMISE_IN_BOUNDS)


def _allsum(v, perms):
    for p in perms:
        v = v + _permute(v, p)
    return v


def _rsqrt(x):
    i = lax.bitcast_convert_type(x, jnp.int32)
    i = jnp.int32(0x5F3759DF) - lax.shift_right_logical(i, 1)
    y = lax.bitcast_convert_type(i, jnp.float32)
    hx = x * 0.5
    for _ in range(3):
        y = y * (1.5 - hx * y * y)
    return y


_VB = 32768
_LOG_VB = 15


@functools.lru_cache(maxsize=None)
def _make_retile(vocab):
    grid = vocab // _VB

    def body(in_ref, out_ref):
        z = in_ref[...].T
        out_ref[:, 0:DIM] = z[0:_VB // 2]
        out_ref[:, DIM:2 * DIM] = z[_VB // 2:_VB]

    return pl.pallas_call(
        body,
        grid=(grid,),
        in_specs=[pl.BlockSpec((DIM, _VB), lambda i: (0, i))],
        out_specs=pl.BlockSpec((_VB // 2, 2 * DIM), lambda i: (i, 0)),
        out_shape=jax.ShapeDtypeStruct((vocab // 2, 2 * DIM), jnp.float32),
    )


@functools.lru_cache(maxsize=None)
def _make_tail(vocab):
    start = (vocab // _VB) * _VB
    n = vocab - start
    half = n // 2
    qrow = start // 2

    def body(tail_ref, acc_ref, out_ref, vout, sem):
        del acc_ref
        z = tail_ref[...].T
        vout[:, 0:DIM] = z[0:half]
        vout[:, DIM:2 * DIM] = z[half:n]
        cp = pltpu.make_async_copy(
            vout, out_ref.at[pl.ds(qrow, half)], sem)
        cp.start()
        cp.wait()

    return pl.pallas_call(
        body,
        in_specs=[pl.BlockSpec((DIM, n), lambda: (0, 0)),
                  pl.BlockSpec(memory_space=pl.ANY)],
        out_specs=pl.BlockSpec(memory_space=pl.ANY),
        scratch_shapes=[pltpu.VMEM((half, 2 * DIM), jnp.float32),
                        pltpu.SemaphoreType.DMA],
        out_shape=jax.ShapeDtypeStruct((vocab // 2, 2 * DIM), jnp.float32),
        input_output_aliases={1: 0},
    )


@functools.lru_cache(maxsize=None)
def _make_sc_kernel(batch, seq, vocab):
    tstart = (vocab // _VB) * _VB
    thalf = (vocab - tstart) // 2
    info = plsc.get_sparse_core_info()
    nc, ns = info.num_cores, info.num_subcores
    nw = nc * ns
    b_per_w = batch // nw
    n_chunks = b_per_w // NB
    rows = seq * NB
    mesh = plsc.VectorSubcoreMesh(core_axis_name="c", subcore_axis_name="s")

    @functools.partial(
        pl.kernel,
        mesh=mesh,
        compiler_params=pltpu.CompilerParams(use_tc_tiling_on_sc=False),
        out_type=jax.ShapeDtypeStruct((seq, batch, DIM), jnp.float32),
        scratch_types=[
            pltpu.VMEM((seq, NB), jnp.int32),
            pltpu.VMEM((seq, NB), jnp.int32),
            pltpu.VMEM((seq, NB, DIM), jnp.float32),
            pltpu.VMEM((seq, NB, DIM), jnp.float32),
            pltpu.VMEM((seq, DIM), jnp.float32),
            pltpu.VMEM((DIM,), jnp.float32),
            pltpu.VMEM((DIM,), jnp.float32),
            pltpu.SemaphoreType.DMA,
            pltpu.SemaphoreType.DMA,
            pltpu.SemaphoreType.DMA,
            pltpu.SemaphoreType.DMA,
        ],
    )
    def k(tok_hbm, table_hbm, pos_hbm, gamma_hbm, beta_hbm, out_hbm,
          idx0, idx1, rows0, rows1, pos_v, gam_v, bet_v,
          gsem0, gsem1, osem0, osem1):
        idx = [idx0, idx1]
        rbuf = [rows0, rows1]
        gsem = [gsem0, gsem1]
        osem = [osem0, osem1]
        wid = lax.axis_index("s") * nc + lax.axis_index("c")
        bbase = wid * b_per_w

        pltpu.sync_copy(pos_hbm.at[pl.ds(0, seq)], pos_v)
        pltpu.sync_copy(gamma_hbm, gam_v)
        pltpu.sync_copy(beta_hbm, bet_v)
        gammas = [gam_v[pl.ds(LANES * j, LANES)] for j in range(NVR)]
        betas = [bet_v[pl.ds(LANES * j, LANES)] for j in range(NVR)]
        lane = lax.iota(jnp.int32, LANES)
        perms = [lane ^ d for d in (1, 2, 4, 8)]

        def fire_gathers(c):
            b = c % 2
            pltpu.sync_copy(
                tok_hbm.at[:, pl.ds(bbase + c * NB, NB)], idx[b])
            for s in range(seq):
                v = idx[b][s, :]
                q = ((v & ~(_VB - 1))
                     + ((v & (_VB // 2 - 1)) << 1)
                     + ((v >> (_LOG_VB - 1)) & 1))
                if tstart < vocab:
                    w = v - tstart
                    hi = w >= thalf
                    qt = (tstart + ((w - jnp.where(hi, thalf, 0)) << 1)
                          + jnp.where(hi, 1, 0))
                    q = jnp.where(v >= tstart, qt, q)
                idx[b][s, :] = q
            return [
                pltpu.async_copy(table_hbm.at[idx[b].at[s]],
                                 rbuf[b].at[s], gsem[b])
                for s in range(seq)
            ]

        gcps = {0: fire_gathers(0)}
        ocps = {}
        for c in range(n_chunks):
            b = c % 2
            if c + 1 < n_chunks:
                if c - 1 >= 0:
                    ocps[c - 1].wait()
                gcps[c + 1] = fire_gathers(c + 1)
            for cp in gcps.pop(c):
                cp.wait()

            rows_b = rbuf[c % 2]

            @plsc.parallel_loop(0, rows, unroll=UNROLL)
            def _(i):
                s = lax.shift_right_logical(i, LOG_NB)
                r = lax.bitwise_and(i, NB - 1)
                xs = [rows_b[s, r, pl.ds(LANES * j, LANES)]
                      + pos_v[s, pl.ds(LANES * j, LANES)]
                      for j in range(NVR)]
                tot = (xs[0] + xs[1]) + (xs[2] + xs[3])
                mean = _allsum(tot, perms) * (1.0 / DIM)
                cs = [x - mean for x in xs]
                sq = (cs[0] * cs[0] + cs[1] * cs[1]) + (cs[2] * cs[2]
                                                        + cs[3] * cs[3])
                var = _allsum(sq, perms) * (1.0 / DIM)
                rstd = _rsqrt(var + EPS)
                for j in range(NVR):
                    rows_b[s, r, pl.ds(LANES * j, LANES)] = (
                        cs[j] * rstd * gammas[j] + betas[j])

            ocps[c] = pltpu.async_copy(
                rows_b, out_hbm.at[:, pl.ds(bbase + c * NB, NB)],
                osem[c % 2])
        for c in (n_chunks - 2, n_chunks - 1):
            ocps[c].wait()

    return k


def kernel(tokens, table, pos, gamma, beta):
    batch, seq = tokens.shape
    vocab = table.shape[0]
    tab_t = table.T
    tab2 = _make_retile(vocab)(tab_t)
    if vocab % _VB:
        tab2 = _make_tail(vocab)(tab_t[:, (vocab // _VB) * _VB:], tab2)
    tab_lin = tab2.reshape(vocab, DIM)
    out = _make_sc_kernel(batch, seq, vocab)(tokens.T.astype(jnp.int32),
                                             tab_lin, pos[0], gamma, beta)
    return out.transpose(1, 0, 2)
